# Initial kernel scaffold; baseline (speedup 1.0000x reference)
import jax
import jax.numpy as jnp
from jax.experimental import pallas as pl


def kernel(activation):
    raise NotImplementedError("write your pallas kernel here")


# speedup vs baseline: 2.0562x; 2.0562x over previous
import functools

import jax
import jax.numpy as jnp
from jax import lax
from jax.experimental import pallas as pl
from jax.experimental.pallas import tpu as pltpu
from jax.experimental.pallas import tpu_sc as plsc

_N, _C, _H, _W = 4, 128, 224, 224
_IMGS = _N * _C
_PIX = _H * _W
_NTEC = 32
_PER = _IMGS // _NTEC

_mesh = plsc.VectorSubcoreMesh(core_axis_name="c", subcore_axis_name="s")


@functools.partial(
    pl.kernel,
    out_type=jax.ShapeDtypeStruct((_IMGS, _PIX), jnp.float32),
    mesh=_mesh,
    scratch_types=[pltpu.VMEM((_PIX,), jnp.float32)],
)
def _block_relu(x_hbm, y_hbm, buf):
    g = lax.axis_index("s") * 2 + lax.axis_index("c")
    typ = g % 8

    lane = lax.iota(jnp.int32, 16)
    e0 = lane & -2
    e1 = e0 | 1
    q0 = lane & -4
    q1 = q0 | 1
    q2 = q0 | 2
    q3 = q0 | 3

    def gat(v, idx):
        return v.at[idx].get(mode="promise_in_bounds")

    def do_2x2():
        def row(h, carry):
            base = pl.multiple_of(h * (2 * _W), 16)
            for j in range(_W // 16):
                o0 = base + j * 16
                o1 = o0 + _W
                a = buf[pl.ds(o0, 16)]
                b = buf[pl.ds(o1, 16)]
                t = a + b
                s = gat(t, e0) + gat(t, e1)
                keep = s >= 0.0
                buf[pl.ds(o0, 16)] = jnp.where(keep, a, 0.0)
                buf[pl.ds(o1, 16)] = jnp.where(keep, b, 0.0)
            return carry
        lax.fori_loop(0, _H // 2, row, 0)

    def do_4x4():
        def row(h, carry):
            base = pl.multiple_of(h * (4 * _W), 16)
            for j in range(_W // 16):
                o0 = base + j * 16
                r0 = buf[pl.ds(o0, 16)]
                r1 = buf[pl.ds(o0 + _W, 16)]
                r2 = buf[pl.ds(o0 + 2 * _W, 16)]
                r3 = buf[pl.ds(o0 + 3 * _W, 16)]
                t = (r0 + r1) + (r2 + r3)
                s = (gat(t, q0) + gat(t, q1)) + (gat(t, q2) + gat(t, q3))
                keep = s >= 0.0
                buf[pl.ds(o0, 16)] = jnp.where(keep, r0, 0.0)
                buf[pl.ds(o0 + _W, 16)] = jnp.where(keep, r1, 0.0)
                buf[pl.ds(o0 + 2 * _W, 16)] = jnp.where(keep, r2, 0.0)
                buf[pl.ds(o0 + 3 * _W, 16)] = jnp.where(keep, r3, 0.0)
            return carry
        lax.fori_loop(0, _H // 4, row, 0)

    def img_body(i, carry):
        img = g * _PER + i
        pltpu.sync_copy(x_hbm.at[img], buf)
        pl.when(typ < 4)(do_2x2)
        pl.when((typ >= 4) & (typ < 7))(do_4x4)
        pltpu.sync_copy(buf, y_hbm.at[img])
        return carry

    lax.fori_loop(0, _PER, img_body, 0)


def kernel(activation):
    x = activation.reshape(_IMGS, _PIX)
    y = _block_relu(x)
    return y.reshape(_N, _C, _H, _W)
